# baseline (speedup 1.0000x reference)
import functools

import jax
import jax.numpy as jnp
import numpy as np
from jax import lax
from jax.experimental import pallas as pl
from jax.experimental.pallas import tpu as pltpu
from jax.experimental.pallas import tpu_sc as plsc

_NUM_CLUSTERS = 4096
_K = 1024
_C = 256
_B = 32
_H = 100
_W = 100
_MAX_DIST = 50.0
_NUM_BINS = 50
_HID = 512


_BB = 2

_NW = 32
_PPW = (_B * _C) // _NW
_PBLK = 4
_HW = _H * _W


def _sc_gather_body(bev_hbm, idx_hbm, wgt_hbm, out_hbm,
                    idx_v, wgt_v, buf_v, outb_v, sem):
    wid = lax.axis_index("s") * 2 + lax.axis_index("c")
    base = wid * _PPW
    pltpu.sync_copy(idx_hbm, idx_v)
    pltpu.sync_copy(wgt_hbm, wgt_v)

    def block(g, carry):
        row = base + g * _PBLK
        pltpu.async_copy(bev_hbm.at[pl.ds(row * _HW, _PBLK * _HW)],
                         buf_v, sem).wait()

        def chunk(c, carry2):
            sl = pl.ds(c * 16, 16)
            i00 = idx_v[sl]
            i01 = idx_v[pl.ds(_K + c * 16, 16)]
            i10 = idx_v[pl.ds(2 * _K + c * 16, 16)]
            i11 = idx_v[pl.ds(3 * _K + c * 16, 16)]
            du = wgt_v[sl]
            omdu = wgt_v[pl.ds(_K + c * 16, 16)]
            dv = wgt_v[pl.ds(2 * _K + c * 16, 16)]
            omdv = wgt_v[pl.ds(3 * _K + c * 16, 16)]
            for p in range(_PBLK):
                off = jnp.full((16,), p * _HW, jnp.int32)
                f00 = plsc.load_gather(buf_v, [i00 + off])
                f01 = plsc.load_gather(buf_v, [i01 + off])
                f10 = plsc.load_gather(buf_v, [i10 + off])
                f11 = plsc.load_gather(buf_v, [i11 + off])
                top = f00 * omdu + f01 * du
                bot = f10 * omdu + f11 * du
                outb_v[pl.ds(p * _K + c * 16, 16)] = top * omdv + bot * dv
            return carry2

        lax.fori_loop(0, _K // 16, chunk, 0)
        pltpu.sync_copy(outb_v, out_hbm.at[pl.ds(row * _K, _PBLK * _K)])
        return carry

    lax.fori_loop(0, _PPW // _PBLK, block, 0)


@jax.jit
def _sc_gather(bev2, idx4, wgt4):
    mesh = plsc.VectorSubcoreMesh(core_axis_name="c", subcore_axis_name="s")
    out = pl.kernel(
        _sc_gather_body,
        mesh=mesh,
        out_type=jax.ShapeDtypeStruct((_B * _C * _K,), jnp.float32),
        scratch_types=[
            pltpu.VMEM((4 * _K,), jnp.int32),
            pltpu.VMEM((4 * _K,), jnp.float32),
            pltpu.VMEM((_PBLK * _HW,), jnp.float32),
            pltpu.VMEM((_PBLK * _K,), jnp.float32),
            pltpu.SemaphoreType.DMA,
        ],
        compiler_params=pltpu.CompilerParams(needs_layout_passes=False),
    )(bev2, idx4, wgt4)
    return out


def _score_sort_body(ego_ref, bins_ref, feat_ref, goals_ref, w1_ref, b1_ref,
                     w2_ref, b2_ref, selg_ref, sels_ref):
    pid = pl.program_id(0)
    goals = goals_ref[...]
    jr = jax.lax.broadcasted_iota(jnp.int32, (_K, _K), 0)
    jc = jax.lax.broadcasted_iota(jnp.int32, (_K, _K), 1)
    tri = jc < jr
    iota_bins = jax.lax.broadcasted_iota(jnp.int32, (_K, _NUM_BINS), 1)

    for i in range(_BB):
        b = pid * _BB + i
        h = jax.nn.relu(
            jax.lax.dot_general(feat_ref[i], w1_ref[...],
                                (((0,), (0,)), ((), ())))
            + b1_ref[...])
        driv = (jax.lax.dot_general(h, w2_ref[...], (((1,), (0,)), ((), ())))
                + b2_ref[0, 0])
        ex = ego_ref[b, 0]
        ey = ego_ref[b, 1]
        dx = goals[:, 0:1] - ex
        dy = goals[:, 1:2] - ey
        dist = jnp.sqrt(dx * dx + dy * dy + 1e-8)
        bin_idx = jnp.clip((dist / _MAX_DIST * _NUM_BINS).astype(jnp.int32),
                           0, _NUM_BINS - 1)
        bsel = jnp.where(bin_idx == iota_bins, bins_ref[...], 0.0)
        dist_score = jnp.sum(bsel, axis=1, keepdims=True)

        s_col = dist_score + driv
        s_row = jnp.transpose(s_col)

        beats = (s_row > s_col) | ((s_row == s_col) & tri)
        rank_col = jnp.sum(jnp.where(beats, 1.0, 0.0), axis=1,
                           keepdims=True)
        rank_row = jnp.transpose(rank_col)

        p_mat = jnp.where(jr == rank_row.astype(jnp.int32), 1.0, 0.0)
        src = jnp.concatenate([goals, s_col], axis=1)
        out3 = jax.lax.dot_general(
            p_mat, src, (((1,), (0,)), ((), ())),
            precision=jax.lax.Precision.HIGHEST)
        selg_ref[i] = out3[:, 0:2]
        sels_ref[i] = jnp.transpose(out3[:, 2:3])


@jax.jit
def _score_sort(ego_state, feat, goals, distance_bin_scores, w1, b1, w2, b2):
    grid = (_B // _BB,)
    selg, sels = pl.pallas_call(
        _score_sort_body,
        grid=grid,
        in_specs=[
            pl.BlockSpec(memory_space=pltpu.SMEM),
            pl.BlockSpec((1, _NUM_BINS), lambda b: (0, 0)),
            pl.BlockSpec((_BB, _C, _K), lambda b: (b, 0, 0)),
            pl.BlockSpec((_K, 2), lambda b: (0, 0)),
            pl.BlockSpec((_C, _HID), lambda b: (0, 0)),
            pl.BlockSpec((1, _HID), lambda b: (0, 0)),
            pl.BlockSpec((_HID, 1), lambda b: (0, 0)),
            pl.BlockSpec(memory_space=pltpu.SMEM),
        ],
        out_specs=[
            pl.BlockSpec((_BB, _K, 2), lambda b: (b, 0, 0)),
            pl.BlockSpec((_BB, 1, _K), lambda b: (b, 0, 0)),
        ],
        out_shape=[
            jax.ShapeDtypeStruct((_B, _K, 2), jnp.float32),
            jax.ShapeDtypeStruct((_B, 1, _K), jnp.float32),
        ],
        compiler_params=pltpu.CompilerParams(
            dimension_semantics=("arbitrary",)),
    )(ego_state, distance_bin_scores.reshape(1, _NUM_BINS), feat, goals,
      w1, b1.reshape(1, _HID), w2, b2.reshape(1, 1))
    return selg, sels.reshape(_B, _K)


def kernel(ego_state, bev_features, cluster_centers, distance_bin_scores,
           w1, b1, w2, b2):
    perm = jax.random.permutation(jax.random.key(42), _NUM_CLUSTERS)[:_K]
    goals = jnp.take(cluster_centers, perm, axis=0)
    u = jnp.clip((goals[:, 0] + _MAX_DIST) / (2.0 * _MAX_DIST) * (_W - 1),
                 0.0, _W - 1.0)
    v = jnp.clip((goals[:, 1] + _MAX_DIST) / (2.0 * _MAX_DIST) * (_H - 1),
                 0.0, _H - 1.0)
    u0 = jnp.floor(u).astype(jnp.int32)
    v0 = jnp.floor(v).astype(jnp.int32)
    u1 = jnp.clip(u0 + 1, 0, _W - 1)
    v1 = jnp.clip(v0 + 1, 0, _H - 1)
    u0c = jnp.clip(u0, 0, _W - 1)
    v0c = jnp.clip(v0, 0, _H - 1)
    du = u - u0c.astype(u.dtype)
    dv = v - v0c.astype(v.dtype)
    idx4 = jnp.concatenate([v0c * _W + u0c, v0c * _W + u1,
                            v1 * _W + u0c, v1 * _W + u1])
    wgt4 = jnp.concatenate([du, 1.0 - du, dv, 1.0 - dv])
    featT = _sc_gather(bev_features.reshape(_B * _C * _HW), idx4, wgt4)

    selected_goals, selected_scores = _score_sort(
        ego_state, featT.reshape(_B, _C, _K), goals, distance_bin_scores,
        w1, b1, w2, b2)
    candidate_goals = jnp.broadcast_to(goals[None, :, :], (_B, _K, 2))
    return selected_goals, selected_scores, candidate_goals


# speedup vs baseline: 2.4225x; 2.4225x over previous
import functools

import jax
import jax.numpy as jnp
import numpy as np
from jax import lax
from jax.experimental import pallas as pl
from jax.experimental.pallas import tpu as pltpu
from jax.experimental.pallas import tpu_sc as plsc

_NUM_CLUSTERS = 4096
_K = 1024
_C = 256
_B = 32
_H = 100
_W = 100
_MAX_DIST = 50.0
_NUM_BINS = 50
_HID = 512


_BB = 4

_NW = 32
_PPW = (_B * _C) // _NW
_PBLK = 4
_HW = _H * _W


def _sc_gather_body(bev_hbm, idx_hbm, wgt_hbm, out_hbm,
                    idx_v, wgt_v, buf_v, outb_v, sem):
    wid = lax.axis_index("s") * 2 + lax.axis_index("c")
    base = wid * _PPW
    pltpu.sync_copy(idx_hbm, idx_v)
    pltpu.sync_copy(wgt_hbm, wgt_v)

    def block(g, carry):
        row = base + g * _PBLK
        pltpu.async_copy(bev_hbm.at[pl.ds(row * _HW, _PBLK * _HW)],
                         buf_v, sem).wait()

        def chunk(c, carry2):
            sl = pl.ds(c * 16, 16)
            i00 = idx_v[sl]
            i01 = idx_v[pl.ds(_K + c * 16, 16)]
            i10 = idx_v[pl.ds(2 * _K + c * 16, 16)]
            i11 = idx_v[pl.ds(3 * _K + c * 16, 16)]
            du = wgt_v[sl]
            omdu = wgt_v[pl.ds(_K + c * 16, 16)]
            dv = wgt_v[pl.ds(2 * _K + c * 16, 16)]
            omdv = wgt_v[pl.ds(3 * _K + c * 16, 16)]
            for p in range(_PBLK):
                off = jnp.full((16,), p * _HW, jnp.int32)
                f00 = plsc.load_gather(buf_v, [i00 + off])
                f01 = plsc.load_gather(buf_v, [i01 + off])
                f10 = plsc.load_gather(buf_v, [i10 + off])
                f11 = plsc.load_gather(buf_v, [i11 + off])
                top = f00 * omdu + f01 * du
                bot = f10 * omdu + f11 * du
                outb_v[pl.ds(p * _K + c * 16, 16)] = top * omdv + bot * dv
            return carry2

        lax.fori_loop(0, _K // 16, chunk, 0)
        pltpu.sync_copy(outb_v, out_hbm.at[pl.ds(row * _K, _PBLK * _K)])
        return carry

    lax.fori_loop(0, _PPW // _PBLK, block, 0)


@jax.jit
def _sc_gather(bev2, idx4, wgt4):
    mesh = plsc.VectorSubcoreMesh(core_axis_name="c", subcore_axis_name="s")
    out = pl.kernel(
        _sc_gather_body,
        mesh=mesh,
        out_type=jax.ShapeDtypeStruct((_B * _C * _K,), jnp.float32),
        scratch_types=[
            pltpu.VMEM((4 * _K,), jnp.int32),
            pltpu.VMEM((4 * _K,), jnp.float32),
            pltpu.VMEM((_PBLK * _HW,), jnp.float32),
            pltpu.VMEM((_PBLK * _K,), jnp.float32),
            pltpu.SemaphoreType.DMA,
        ],
        compiler_params=pltpu.CompilerParams(needs_layout_passes=False),
    )(bev2, idx4, wgt4)
    return out


def _score_sort_body(ego_ref, bins_ref, feat_ref, goals_ref, w1_ref, b1_ref,
                     w2_ref, b2_ref, selg_ref, sels_ref):
    pid = pl.program_id(0)
    goals = goals_ref[...]
    jr = jax.lax.broadcasted_iota(jnp.int32, (_K, _K), 0)
    jc = jax.lax.broadcasted_iota(jnp.int32, (_K, _K), 1)
    tri = jc < jr
    iota_bins = jax.lax.broadcasted_iota(jnp.int32, (_K, _NUM_BINS), 1)

    for i in range(_BB):
        b = pid * _BB + i
        h = jax.nn.relu(
            jax.lax.dot_general(feat_ref[i], w1_ref[...],
                                (((0,), (0,)), ((), ())))
            + b1_ref[...])
        driv = (jax.lax.dot_general(h, w2_ref[...], (((1,), (0,)), ((), ())))
                + b2_ref[0, 0])
        ex = ego_ref[b, 0]
        ey = ego_ref[b, 1]
        dx = goals[:, 0:1] - ex
        dy = goals[:, 1:2] - ey
        dist = jnp.sqrt(dx * dx + dy * dy + 1e-8)
        bin_idx = jnp.clip((dist / _MAX_DIST * _NUM_BINS).astype(jnp.int32),
                           0, _NUM_BINS - 1)
        bsel = jnp.where(bin_idx == iota_bins, bins_ref[...], 0.0)
        dist_score = jnp.sum(bsel, axis=1, keepdims=True)

        s_col = dist_score + driv
        s_row = jnp.transpose(s_col)

        beats = (s_row > s_col) | ((s_row == s_col) & tri)
        rank_col = jnp.sum(jnp.where(beats, 1.0, 0.0), axis=1,
                           keepdims=True)
        rank_row = jnp.transpose(rank_col)

        p_mat = jnp.where(jr == rank_row.astype(jnp.int32), 1.0, 0.0)
        src = jnp.concatenate([goals, s_col], axis=1)
        out3 = jax.lax.dot_general(
            p_mat, src, (((1,), (0,)), ((), ())),
            precision=jax.lax.Precision.HIGHEST)
        selg_ref[i] = out3[:, 0:2]
        sels_ref[i] = jnp.transpose(out3[:, 2:3])


@jax.jit
def _score_sort(ego_state, feat, goals, distance_bin_scores, w1, b1, w2, b2):
    grid = (_B // _BB,)
    selg, sels = pl.pallas_call(
        _score_sort_body,
        grid=grid,
        in_specs=[
            pl.BlockSpec(memory_space=pltpu.SMEM),
            pl.BlockSpec((1, _NUM_BINS), lambda b: (0, 0)),
            pl.BlockSpec((_BB, _C, _K), lambda b: (b, 0, 0)),
            pl.BlockSpec((_K, 2), lambda b: (0, 0)),
            pl.BlockSpec((_C, _HID), lambda b: (0, 0)),
            pl.BlockSpec((1, _HID), lambda b: (0, 0)),
            pl.BlockSpec((_HID, 1), lambda b: (0, 0)),
            pl.BlockSpec(memory_space=pltpu.SMEM),
        ],
        out_specs=[
            pl.BlockSpec((_BB, _K, 2), lambda b: (b, 0, 0)),
            pl.BlockSpec((_BB, 1, _K), lambda b: (b, 0, 0)),
        ],
        out_shape=[
            jax.ShapeDtypeStruct((_B, _K, 2), jnp.float32),
            jax.ShapeDtypeStruct((_B, 1, _K), jnp.float32),
        ],
        compiler_params=pltpu.CompilerParams(
            dimension_semantics=("arbitrary",)),
    )(ego_state, distance_bin_scores.reshape(1, _NUM_BINS), feat, goals,
      w1, b1.reshape(1, _HID), w2, b2.reshape(1, 1))
    return selg, sels.reshape(_B, _K)


def kernel(ego_state, bev_features, cluster_centers, distance_bin_scores,
           w1, b1, w2, b2):
    perm = jax.random.permutation(jax.random.key(42), _NUM_CLUSTERS)[:_K]
    goals = jnp.take(cluster_centers, perm, axis=0)
    u = jnp.clip((goals[:, 0] + _MAX_DIST) / (2.0 * _MAX_DIST) * (_W - 1),
                 0.0, _W - 1.0)
    v = jnp.clip((goals[:, 1] + _MAX_DIST) / (2.0 * _MAX_DIST) * (_H - 1),
                 0.0, _H - 1.0)
    u0 = jnp.floor(u).astype(jnp.int32)
    v0 = jnp.floor(v).astype(jnp.int32)
    u1 = jnp.clip(u0 + 1, 0, _W - 1)
    v1 = jnp.clip(v0 + 1, 0, _H - 1)
    u0c = jnp.clip(u0, 0, _W - 1)
    v0c = jnp.clip(v0, 0, _H - 1)
    du = u - u0c.astype(u.dtype)
    dv = v - v0c.astype(v.dtype)
    f00 = bev_features[:, :, v0c, u0c]
    f01 = bev_features[:, :, v0c, u1]
    f10 = bev_features[:, :, v1, u0c]
    f11 = bev_features[:, :, v1, u1]
    dub = du[None, None, :]
    dvb = dv[None, None, :]
    top = f00 * (1.0 - dub) + f01 * dub
    bot = f10 * (1.0 - dub) + f11 * dub
    featT = top * (1.0 - dvb) + bot * dvb

    selected_goals, selected_scores = _score_sort(
        ego_state, featT, goals, distance_bin_scores, w1, b1, w2, b2)
    candidate_goals = jnp.broadcast_to(goals[None, :, :], (_B, _K, 2))
    return selected_goals, selected_scores, candidate_goals


# speedup vs baseline: 2.5776x; 1.0640x over previous
import functools

import jax
import jax.numpy as jnp
import numpy as np
from jax import lax
from jax.experimental import pallas as pl
from jax.experimental.pallas import tpu as pltpu
from jax.experimental.pallas import tpu_sc as plsc

_NUM_CLUSTERS = 4096
_K = 1024
_C = 256
_B = 32
_H = 100
_W = 100
_MAX_DIST = 50.0
_NUM_BINS = 50
_HID = 512


_BB = 2

_NW = 32
_PPW = (_B * _C) // _NW
_PBLK = 4
_HW = _H * _W


def _sc_gather_body(bev_hbm, idx_hbm, wgt_hbm, out_hbm,
                    idx_v, wgt_v, buf_v, outb_v, sem):
    wid = lax.axis_index("s") * 2 + lax.axis_index("c")
    base = wid * _PPW
    pltpu.sync_copy(idx_hbm, idx_v)
    pltpu.sync_copy(wgt_hbm, wgt_v)

    def block(g, carry):
        row = base + g * _PBLK
        pltpu.async_copy(bev_hbm.at[pl.ds(row * _HW, _PBLK * _HW)],
                         buf_v, sem).wait()

        def chunk(c, carry2):
            sl = pl.ds(c * 16, 16)
            i00 = idx_v[sl]
            i01 = idx_v[pl.ds(_K + c * 16, 16)]
            i10 = idx_v[pl.ds(2 * _K + c * 16, 16)]
            i11 = idx_v[pl.ds(3 * _K + c * 16, 16)]
            du = wgt_v[sl]
            omdu = wgt_v[pl.ds(_K + c * 16, 16)]
            dv = wgt_v[pl.ds(2 * _K + c * 16, 16)]
            omdv = wgt_v[pl.ds(3 * _K + c * 16, 16)]
            for p in range(_PBLK):
                off = jnp.full((16,), p * _HW, jnp.int32)
                f00 = plsc.load_gather(buf_v, [i00 + off])
                f01 = plsc.load_gather(buf_v, [i01 + off])
                f10 = plsc.load_gather(buf_v, [i10 + off])
                f11 = plsc.load_gather(buf_v, [i11 + off])
                top = f00 * omdu + f01 * du
                bot = f10 * omdu + f11 * du
                outb_v[pl.ds(p * _K + c * 16, 16)] = top * omdv + bot * dv
            return carry2

        lax.fori_loop(0, _K // 16, chunk, 0)
        pltpu.sync_copy(outb_v, out_hbm.at[pl.ds(row * _K, _PBLK * _K)])
        return carry

    lax.fori_loop(0, _PPW // _PBLK, block, 0)


@jax.jit
def _sc_gather(bev2, idx4, wgt4):
    mesh = plsc.VectorSubcoreMesh(core_axis_name="c", subcore_axis_name="s")
    out = pl.kernel(
        _sc_gather_body,
        mesh=mesh,
        out_type=jax.ShapeDtypeStruct((_B * _C * _K,), jnp.float32),
        scratch_types=[
            pltpu.VMEM((4 * _K,), jnp.int32),
            pltpu.VMEM((4 * _K,), jnp.float32),
            pltpu.VMEM((_PBLK * _HW,), jnp.float32),
            pltpu.VMEM((_PBLK * _K,), jnp.float32),
            pltpu.SemaphoreType.DMA,
        ],
        compiler_params=pltpu.CompilerParams(needs_layout_passes=False),
    )(bev2, idx4, wgt4)
    return out


def _score_sort_body(ego_ref, bins_ref, f00_ref, f01_ref, f10_ref, f11_ref,
                     du_ref, dv_ref, goals_ref, w1_ref, b1_ref,
                     w2_ref, b2_ref, selg_ref, sels_ref):
    pid = pl.program_id(0)
    goals = goals_ref[...]
    du = du_ref[...]
    dv = dv_ref[...]
    omdu = 1.0 - du
    omdv = 1.0 - dv
    jr = jax.lax.broadcasted_iota(jnp.int32, (_K, _K), 0)
    jc = jax.lax.broadcasted_iota(jnp.int32, (_K, _K), 1)
    tri = jc < jr
    iota_bins = jax.lax.broadcasted_iota(jnp.int32, (_K, _NUM_BINS), 1)

    for i in range(_BB):
        b = pid * _BB + i
        top = f00_ref[i] * omdu + f01_ref[i] * du
        bot = f10_ref[i] * omdu + f11_ref[i] * du
        feat_t = top * omdv + bot * dv
        h = jax.nn.relu(
            jax.lax.dot_general(feat_t, w1_ref[...],
                                (((0,), (0,)), ((), ())))
            + b1_ref[...])
        driv = (jax.lax.dot_general(h, w2_ref[...], (((1,), (0,)), ((), ())))
                + b2_ref[0, 0])
        ex = ego_ref[b, 0]
        ey = ego_ref[b, 1]
        dx = goals[:, 0:1] - ex
        dy = goals[:, 1:2] - ey
        dist = jnp.sqrt(dx * dx + dy * dy + 1e-8)
        bin_idx = jnp.clip((dist / _MAX_DIST * _NUM_BINS).astype(jnp.int32),
                           0, _NUM_BINS - 1)
        bsel = jnp.where(bin_idx == iota_bins, bins_ref[...], 0.0)
        dist_score = jnp.sum(bsel, axis=1, keepdims=True)

        s_col = dist_score + driv
        s_row = jnp.transpose(s_col)

        beats = (s_row > s_col) | ((s_row == s_col) & tri)
        rank_col = jnp.sum(jnp.where(beats, 1.0, 0.0), axis=1,
                           keepdims=True)
        rank_row = jnp.transpose(rank_col)

        p_mat = jnp.where(jr == rank_row.astype(jnp.int32), 1.0, 0.0)
        src = jnp.concatenate([goals, s_col], axis=1)
        out3 = jax.lax.dot_general(
            p_mat, src, (((1,), (0,)), ((), ())),
            precision=jax.lax.Precision.HIGHEST)
        selg_ref[i] = out3[:, 0:2]
        sels_ref[i] = jnp.transpose(out3[:, 2:3])


@jax.jit
def _score_sort(ego_state, f00, f01, f10, f11, du, dv, goals,
                distance_bin_scores, w1, b1, w2, b2):
    grid = (_B // _BB,)
    selg, sels = pl.pallas_call(
        _score_sort_body,
        grid=grid,
        in_specs=[
            pl.BlockSpec(memory_space=pltpu.SMEM),
            pl.BlockSpec((1, _NUM_BINS), lambda b: (0, 0)),
            pl.BlockSpec((_BB, _C, _K), lambda b: (b, 0, 0)),
            pl.BlockSpec((_BB, _C, _K), lambda b: (b, 0, 0)),
            pl.BlockSpec((_BB, _C, _K), lambda b: (b, 0, 0)),
            pl.BlockSpec((_BB, _C, _K), lambda b: (b, 0, 0)),
            pl.BlockSpec((1, _K), lambda b: (0, 0)),
            pl.BlockSpec((1, _K), lambda b: (0, 0)),
            pl.BlockSpec((_K, 2), lambda b: (0, 0)),
            pl.BlockSpec((_C, _HID), lambda b: (0, 0)),
            pl.BlockSpec((1, _HID), lambda b: (0, 0)),
            pl.BlockSpec((_HID, 1), lambda b: (0, 0)),
            pl.BlockSpec(memory_space=pltpu.SMEM),
        ],
        out_specs=[
            pl.BlockSpec((_BB, _K, 2), lambda b: (b, 0, 0)),
            pl.BlockSpec((_BB, 1, _K), lambda b: (b, 0, 0)),
        ],
        out_shape=[
            jax.ShapeDtypeStruct((_B, _K, 2), jnp.float32),
            jax.ShapeDtypeStruct((_B, 1, _K), jnp.float32),
        ],
        compiler_params=pltpu.CompilerParams(
            dimension_semantics=("arbitrary",)),
    )(ego_state, distance_bin_scores.reshape(1, _NUM_BINS),
      f00, f01, f10, f11, du.reshape(1, _K), dv.reshape(1, _K), goals,
      w1, b1.reshape(1, _HID), w2, b2.reshape(1, 1))
    return selg, sels.reshape(_B, _K)


def kernel(ego_state, bev_features, cluster_centers, distance_bin_scores,
           w1, b1, w2, b2):
    perm = jax.random.permutation(jax.random.key(42), _NUM_CLUSTERS)[:_K]
    goals = jnp.take(cluster_centers, perm, axis=0)
    u = jnp.clip((goals[:, 0] + _MAX_DIST) / (2.0 * _MAX_DIST) * (_W - 1),
                 0.0, _W - 1.0)
    v = jnp.clip((goals[:, 1] + _MAX_DIST) / (2.0 * _MAX_DIST) * (_H - 1),
                 0.0, _H - 1.0)
    u0 = jnp.floor(u).astype(jnp.int32)
    v0 = jnp.floor(v).astype(jnp.int32)
    u1 = jnp.clip(u0 + 1, 0, _W - 1)
    v1 = jnp.clip(v0 + 1, 0, _H - 1)
    u0c = jnp.clip(u0, 0, _W - 1)
    v0c = jnp.clip(v0, 0, _H - 1)
    du = u - u0c.astype(u.dtype)
    dv = v - v0c.astype(v.dtype)
    f00 = bev_features[:, :, v0c, u0c]
    f01 = bev_features[:, :, v0c, u1]
    f10 = bev_features[:, :, v1, u0c]
    f11 = bev_features[:, :, v1, u1]

    selected_goals, selected_scores = _score_sort(
        ego_state, f00, f01, f10, f11, du, dv, goals,
        distance_bin_scores, w1, b1, w2, b2)
    candidate_goals = jnp.broadcast_to(goals[None, :, :], (_B, _K, 2))
    return selected_goals, selected_scores, candidate_goals


# speedup vs baseline: 2.5790x; 1.0006x over previous
import functools

import jax
import jax.numpy as jnp
import numpy as np
from jax.experimental import pallas as pl
from jax.experimental.pallas import tpu as pltpu

_NUM_CLUSTERS = 4096
_K = 1024
_C = 256
_B = 32
_H = 100
_W = 100
_MAX_DIST = 50.0
_NUM_BINS = 50
_HID = 512


_BB = 2


def _score_sort_body(ego_ref, bins_ref, f00_ref, f01_ref, f10_ref, f11_ref,
                     du_ref, dv_ref, goals_ref, w1_ref, b1_ref,
                     w2_ref, b2_ref, selg_ref, sels_ref):
    pid = pl.program_id(0)
    goals = goals_ref[...]
    du = du_ref[...]
    dv = dv_ref[...]
    omdu = 1.0 - du
    omdv = 1.0 - dv
    jr = jax.lax.broadcasted_iota(jnp.int32, (_K, _K), 0)
    jc = jax.lax.broadcasted_iota(jnp.int32, (_K, _K), 1)
    tri = jc < jr
    iota_bins = jax.lax.broadcasted_iota(jnp.int32, (_K, _NUM_BINS), 1)

    for i in range(_BB):
        b = pid * _BB + i
        top = f00_ref[i] * omdu + f01_ref[i] * du
        bot = f10_ref[i] * omdu + f11_ref[i] * du
        feat_t = top * omdv + bot * dv
        h = jax.nn.relu(
            jax.lax.dot_general(feat_t, w1_ref[...],
                                (((0,), (0,)), ((), ())))
            + b1_ref[...])
        driv = (jax.lax.dot_general(h, w2_ref[...], (((1,), (0,)), ((), ())))
                + b2_ref[0, 0])
        ex = ego_ref[b, 0]
        ey = ego_ref[b, 1]
        dx = goals[:, 0:1] - ex
        dy = goals[:, 1:2] - ey
        dist = jnp.sqrt(dx * dx + dy * dy + 1e-8)
        bin_idx = jnp.clip((dist / _MAX_DIST * _NUM_BINS).astype(jnp.int32),
                           0, _NUM_BINS - 1)
        bsel = jnp.where(bin_idx == iota_bins, bins_ref[...], 0.0)
        dist_score = jnp.sum(bsel, axis=1, keepdims=True)

        s_col = dist_score + driv
        s_row = jnp.transpose(s_col)

        beats = (s_row > s_col) | ((s_row == s_col) & tri)
        rank_col = jnp.sum(jnp.where(beats, 1.0, 0.0), axis=1,
                           keepdims=True)
        rank_row = jnp.transpose(rank_col)

        p_mat = jnp.where(jr == rank_row.astype(jnp.int32), 1.0, 0.0)
        src = jnp.concatenate([goals, s_col], axis=1)
        out3 = jax.lax.dot_general(
            p_mat, src, (((1,), (0,)), ((), ())),
            precision=jax.lax.Precision.HIGHEST)
        selg_ref[i] = out3[:, 0:2]
        sels_ref[i] = jnp.transpose(out3[:, 2:3])


@jax.jit
def _score_sort(ego_state, f00, f01, f10, f11, du, dv, goals,
                distance_bin_scores, w1, b1, w2, b2):
    grid = (_B // _BB,)
    selg, sels = pl.pallas_call(
        _score_sort_body,
        grid=grid,
        in_specs=[
            pl.BlockSpec(memory_space=pltpu.SMEM),
            pl.BlockSpec((1, _NUM_BINS), lambda b: (0, 0)),
            pl.BlockSpec((_BB, _C, _K), lambda b: (b, 0, 0)),
            pl.BlockSpec((_BB, _C, _K), lambda b: (b, 0, 0)),
            pl.BlockSpec((_BB, _C, _K), lambda b: (b, 0, 0)),
            pl.BlockSpec((_BB, _C, _K), lambda b: (b, 0, 0)),
            pl.BlockSpec((1, _K), lambda b: (0, 0)),
            pl.BlockSpec((1, _K), lambda b: (0, 0)),
            pl.BlockSpec((_K, 2), lambda b: (0, 0)),
            pl.BlockSpec((_C, _HID), lambda b: (0, 0)),
            pl.BlockSpec((1, _HID), lambda b: (0, 0)),
            pl.BlockSpec((_HID, 1), lambda b: (0, 0)),
            pl.BlockSpec(memory_space=pltpu.SMEM),
        ],
        out_specs=[
            pl.BlockSpec((_BB, _K, 2), lambda b: (b, 0, 0)),
            pl.BlockSpec((_BB, 1, _K), lambda b: (b, 0, 0)),
        ],
        out_shape=[
            jax.ShapeDtypeStruct((_B, _K, 2), jnp.float32),
            jax.ShapeDtypeStruct((_B, 1, _K), jnp.float32),
        ],
        compiler_params=pltpu.CompilerParams(
            dimension_semantics=("arbitrary",)),
    )(ego_state, distance_bin_scores.reshape(1, _NUM_BINS),
      f00, f01, f10, f11, du.reshape(1, _K), dv.reshape(1, _K), goals,
      w1, b1.reshape(1, _HID), w2, b2.reshape(1, 1))
    return selg, sels.reshape(_B, _K)


def kernel(ego_state, bev_features, cluster_centers, distance_bin_scores,
           w1, b1, w2, b2):
    perm = jax.random.permutation(jax.random.key(42), _NUM_CLUSTERS)[:_K]
    goals = jnp.take(cluster_centers, perm, axis=0)
    u = jnp.clip((goals[:, 0] + _MAX_DIST) / (2.0 * _MAX_DIST) * (_W - 1),
                 0.0, _W - 1.0)
    v = jnp.clip((goals[:, 1] + _MAX_DIST) / (2.0 * _MAX_DIST) * (_H - 1),
                 0.0, _H - 1.0)
    u0 = jnp.floor(u).astype(jnp.int32)
    v0 = jnp.floor(v).astype(jnp.int32)
    u1 = jnp.clip(u0 + 1, 0, _W - 1)
    v1 = jnp.clip(v0 + 1, 0, _H - 1)
    u0c = jnp.clip(u0, 0, _W - 1)
    v0c = jnp.clip(v0, 0, _H - 1)
    du = u - u0c.astype(u.dtype)
    dv = v - v0c.astype(v.dtype)
    f00 = bev_features[:, :, v0c, u0c]
    f01 = bev_features[:, :, v0c, u1]
    f10 = bev_features[:, :, v1, u0c]
    f11 = bev_features[:, :, v1, u1]

    selected_goals, selected_scores = _score_sort(
        ego_state, f00, f01, f10, f11, du, dv, goals,
        distance_bin_scores, w1, b1, w2, b2)
    candidate_goals = jnp.broadcast_to(goals[None, :, :], (_B, _K, 2))
    return selected_goals, selected_scores, candidate_goals
